# Initial kernel scaffold; baseline (speedup 1.0000x reference)
import jax
import jax.numpy as jnp
from jax.experimental import pallas as pl


def kernel(inputs, token_table, pos_table):
    raise NotImplementedError("write your pallas kernel here")


# speedup vs baseline: 4.2373x; 4.2373x over previous
import functools
import math

import jax
import jax.numpy as jnp
from jax import lax
from jax.experimental import pallas as pl
from jax.experimental.pallas import tpu as pltpu
from jax.experimental.pallas import tpu_sc as plsc

_NC = 2
_NS = 16
_NW = _NC * _NS
_LANES = 16


def _sc_embed(idx2d, token_table, pos_table, *, batch, seq, dim, scale):
  rows_per_w = batch // _NW
  groups_per_row = seq // 100
  mesh = plsc.VectorSubcoreMesh(
      core_axis_name="c", subcore_axis_name="s",
      num_cores=_NC, num_subcores=_NS)

  @functools.partial(
      pl.kernel,
      mesh=mesh,
      out_type=jax.ShapeDtypeStruct((batch, seq, dim), jnp.float32),
      scratch_types=[
          pltpu.VMEM((rows_per_w * groups_per_row, 100), jnp.int32),
          pltpu.VMEM((seq, dim), jnp.float32),
          pltpu.VMEM((seq, dim), jnp.float32),
          pltpu.SemaphoreType.DMA,
      ],
  )
  def k(idx_hbm, table_hbm, pos_hbm, out_hbm, idx_v, rows_v, pos_v, sem):
    wid = lax.axis_index("s") * _NC + lax.axis_index("c")
    gbase = wid * rows_per_w * groups_per_row
    pltpu.sync_copy(pos_hbm, pos_v)
    pltpu.sync_copy(idx_hbm.at[pl.ds(gbase, rows_per_w * groups_per_row)],
                    idx_v)

    def per_row(r, carry):
      cps = []
      for g in range(groups_per_row):
        cps.append(pltpu.async_copy(
            table_hbm.at[idx_v.at[r * groups_per_row + g]],
            rows_v.at[pl.ds(g * 100, 100)], sem))
      for cp in cps:
        cp.wait()

      def fma(l, c):
        for d in range(dim // _LANES):
          sl = pl.ds(d * _LANES, _LANES)
          rows_v[l, sl] = rows_v[l, sl] * scale + pos_v[l, sl]
        return c
      lax.fori_loop(0, seq, fma, 0)

      pltpu.sync_copy(rows_v, out_hbm.at[wid * rows_per_w + r])
      return carry

    lax.fori_loop(0, rows_per_w, per_row, 0)

  return k(idx2d, token_table, pos_table)


def kernel(inputs, token_table, pos_table):
  batch, seq = inputs.shape
  vocab, dim = token_table.shape
  scale = float(math.sqrt(dim))
  idx2d = inputs.reshape(batch * seq // 100, 100)
  return _sc_embed(idx2d, token_table, pos_table,
                   batch=batch, seq=seq, dim=dim, scale=scale)
